# Initial kernel scaffold; baseline (speedup 1.0000x reference)
import jax
import jax.numpy as jnp
from jax.experimental import pallas as pl


def kernel(x, edge_index, batch, W1, b1, W2, b2, W3, b3, Wfc, bfc):
    raise NotImplementedError("write your pallas kernel here")


# speedup vs baseline: 6.6472x; 6.6472x over previous
import functools

import jax
import jax.numpy as jnp
from jax import lax
from jax.experimental import pallas as pl
from jax.experimental.pallas import tpu as pltpu
from jax.experimental.pallas import tpu_sc as plsc

NN = 10000
EE = 320000
FIN = 128
HH = 256
CC = 10
GG = 64

NPAD = 79 * 128
NB = 160
CH = 32
NCH = NB // CH
EPT = NB * 128
EPAD = 16 * EPT
RPT = NPAD // 16

_mesh = plsc.VectorSubcoreMesh(core_axis_name="c", subcore_axis_name="s")

_f32 = jnp.float32
_i32 = jnp.int32


@functools.partial(
    pl.kernel,
    mesh=_mesh,
    out_type=[
        jax.ShapeDtypeStruct((NPAD, 128), _f32),
        jax.ShapeDtypeStruct((NPAD, 128), _f32),
    ],
    scratch_types=[
        pltpu.VMEM((CH, 128), _i32),
        pltpu.VMEM((128,), _i32),
        pltpu.VMEM((128, 128), _f32),
        pltpu.VMEM((128, 128), _f32),
        pltpu.VMEM_SHARED((NPAD, 128), _f32),
        pltpu.SemaphoreType.DMA,
    ],
)
def _deg_kernel(dst_hbm, cnt0, cnt1, dstbuf, didx, onesbuf, zbuf, shared,
                sem):
    cid = lax.axis_index("c")
    sid = lax.axis_index("s")

    def orow(k, carry):
        def ocol(l, c2):
            onesbuf[k, pl.ds(l * 16, 16)] = jnp.ones((16,), _f32)
            zbuf[k, pl.ds(l * 16, 16)] = jnp.zeros((16,), _f32)
            return c2

        lax.fori_loop(0, 8, ocol, 0)
        return carry

    lax.fori_loop(0, 128, orow, 0)
    base = sid * RPT
    for k in range(4):
        pltpu.sync_copy(zbuf, shared.at[pl.ds(base + k * 128, 128)])
    pltpu.sync_copy(zbuf.at[pl.ds(0, RPT - 512)],
                    shared.at[pl.ds(base + 512, RPT - 512)])
    plsc.subcore_barrier()

    def chunk(c, carry):
        row = sid * NCH + c
        pltpu.sync_copy(dst_hbm.at[row], dstbuf)

        def body(j, carry2):
            def stage(l, c2):
                didx[pl.ds(l * 16, 16)] = dstbuf[j, pl.ds(l * 16, 16)]
                return c2

            lax.fori_loop(0, 8, stage, 0)
            pltpu.sync_copy(onesbuf, shared.at[didx], add=True)
            return carry2

        lax.fori_loop(cid * (CH // 2), cid * (CH // 2) + CH // 2, body, 0)
        return carry

    lax.fori_loop(0, NCH, chunk, 0)
    plsc.subcore_barrier()

    def wout(dsthbm):
        for k in range(5):
            rows = 128 if k < 4 else RPT - 512
            off = base + k * 128
            pltpu.sync_copy(shared.at[pl.ds(off, rows)],
                            zbuf.at[pl.ds(0, rows)])
            pltpu.sync_copy(zbuf.at[pl.ds(0, rows)],
                            dsthbm.at[pl.ds(off, rows)])

    @pl.when(cid == 0)
    def _():
        wout(cnt0)

    @pl.when(cid == 1)
    def _():
        wout(cnt1)


@functools.partial(
    pl.kernel,
    mesh=_mesh,
    out_type=[
        jax.ShapeDtypeStruct((NPAD, 128), _f32),
        jax.ShapeDtypeStruct((NPAD, 128), _f32),
    ],
    scratch_types=[
        pltpu.VMEM((CH, 128), _i32),
        pltpu.VMEM((CH, 128), _i32),
        pltpu.VMEM((128,), _i32),
        pltpu.VMEM((128,), _i32),
        pltpu.VMEM((128, 128), _f32),
        pltpu.VMEM_SHARED((NPAD, 128), _f32),
        pltpu.SemaphoreType.DMA,
    ],
)
def _agg_kernel(g0, g1, src_hbm, dst_hbm, acc0, acc1,
                srcbuf, dstbuf, sidx, didx, gbuf, shared, sem):
    cid = lax.axis_index("c")
    sid = lax.axis_index("s")

    def zrow(k, carry):
        gbuf[k >> 3, pl.ds((k & 7) * 16, 16)] = jnp.zeros((16,), _f32)
        return carry

    lax.fori_loop(0, 128 * 8, zrow, 0)
    base = sid * RPT
    for k in range(4):
        pltpu.sync_copy(gbuf, shared.at[pl.ds(base + k * 128, 128)])
    pltpu.sync_copy(gbuf.at[pl.ds(0, RPT - 512)],
                    shared.at[pl.ds(base + 512, RPT - 512)])
    plsc.subcore_barrier()

    def run(gtab):
        def chunk(c, carry):
            row = sid * NCH + c
            pltpu.sync_copy(src_hbm.at[row], srcbuf)
            pltpu.sync_copy(dst_hbm.at[row], dstbuf)

            def body(j, carry2):
                def stage(l, c2):
                    sidx[pl.ds(l * 16, 16)] = srcbuf[j, pl.ds(l * 16, 16)]
                    didx[pl.ds(l * 16, 16)] = dstbuf[j, pl.ds(l * 16, 16)]
                    return c2

                lax.fori_loop(0, 8, stage, 0)
                pltpu.async_copy(gtab.at[sidx], gbuf, sem).wait()
                pltpu.sync_copy(gbuf, shared.at[didx], add=True)
                return carry2

            lax.fori_loop(0, CH, body, 0)
            return carry

        lax.fori_loop(0, NCH, chunk, 0)

    @pl.when(cid == 0)
    def _():
        run(g0)

    @pl.when(cid == 1)
    def _():
        run(g1)

    plsc.subcore_barrier()

    def wout(dsthbm):
        for k in range(5):
            rows = 128 if k < 4 else RPT - 512
            off = base + k * 128
            pltpu.sync_copy(shared.at[pl.ds(off, rows)],
                            gbuf.at[pl.ds(0, rows)])
            pltpu.sync_copy(gbuf.at[pl.ds(0, rows)],
                            dsthbm.at[pl.ds(off, rows)])

    @pl.when(cid == 0)
    def _():
        wout(acc0)

    @pl.when(cid == 1)
    def _():
        wout(acc1)


def _dinv_body(c0_ref, c1_ref, out_ref):
    i = pl.program_id(0)
    cnt = c0_ref[:, :1] + c1_ref[:, :1]
    node = i * 128 + lax.broadcasted_iota(_i32, (128, 1), 0)
    dinv = jnp.where(node < NN, lax.rsqrt(cnt + 1.0), 0.0)
    out_ref[...] = jnp.broadcast_to(dinv, (128, 128))


def _dinv_call(cnt0, cnt1):
    return pl.pallas_call(
        _dinv_body,
        grid=(79,),
        in_specs=[
            pl.BlockSpec((128, 128), lambda i: (i, 0)),
            pl.BlockSpec((128, 128), lambda i: (i, 0)),
        ],
        out_specs=pl.BlockSpec((128, 128), lambda i: (i, 0)),
        out_shape=jax.ShapeDtypeStruct((NPAD, 128), _f32),
    )(cnt0, cnt1)


def _lin1_body(x_ref, w_ref, drep_ref, g0_ref, g1_ref):
    hw = jnp.dot(x_ref[...], w_ref[...], preferred_element_type=_f32)
    d = drep_ref[...]
    g0_ref[...] = hw[:, :128] * d
    g1_ref[...] = hw[:, 128:] * d


def _lin1_call(xp, w, drep):
    return pl.pallas_call(
        _lin1_body,
        grid=(79,),
        in_specs=[
            pl.BlockSpec((128, FIN), lambda i: (i, 0)),
            pl.BlockSpec((FIN, HH), lambda i: (0, 0)),
            pl.BlockSpec((128, 128), lambda i: (i, 0)),
        ],
        out_specs=[
            pl.BlockSpec((128, 128), lambda i: (i, 0)),
            pl.BlockSpec((128, 128), lambda i: (i, 0)),
        ],
        out_shape=[
            jax.ShapeDtypeStruct((NPAD, 128), _f32),
            jax.ShapeDtypeStruct((NPAD, 128), _f32),
        ],
    )(xp, w, drep)


def _mid_body(a0_ref, a1_ref, g0_ref, g1_ref, drep_ref, w_ref, b_ref,
              o0_ref, o1_ref):
    d = drep_ref[...]
    t0 = (a0_ref[...] + g0_ref[...]) * d + b_ref[:, :128]
    t1 = (a1_ref[...] + g1_ref[...]) * d + b_ref[:, 128:]
    h = jnp.maximum(jnp.concatenate([t0, t1], axis=1), 0.0)
    hw = jnp.dot(h, w_ref[...], preferred_element_type=_f32)
    o0_ref[...] = hw[:, :128] * d
    o1_ref[...] = hw[:, 128:] * d


def _mid_call(a0, a1, g0, g1, drep, w, b2d):
    return pl.pallas_call(
        _mid_body,
        grid=(79,),
        in_specs=[
            pl.BlockSpec((128, 128), lambda i: (i, 0)),
            pl.BlockSpec((128, 128), lambda i: (i, 0)),
            pl.BlockSpec((128, 128), lambda i: (i, 0)),
            pl.BlockSpec((128, 128), lambda i: (i, 0)),
            pl.BlockSpec((128, 128), lambda i: (i, 0)),
            pl.BlockSpec((HH, HH), lambda i: (0, 0)),
            pl.BlockSpec((1, HH), lambda i: (0, 0)),
        ],
        out_specs=[
            pl.BlockSpec((128, 128), lambda i: (i, 0)),
            pl.BlockSpec((128, 128), lambda i: (i, 0)),
        ],
        out_shape=[
            jax.ShapeDtypeStruct((NPAD, 128), _f32),
            jax.ShapeDtypeStruct((NPAD, 128), _f32),
        ],
    )(a0, a1, g0, g1, drep, w, b2d)


def _pool_body(a0_ref, a1_ref, g0_ref, g1_ref, drep_ref, b_ref, bat_ref,
               wfc_ref, bfc_ref, out_ref, sums, cnts):
    i = pl.program_id(0)
    d = drep_ref[...]
    t0 = (a0_ref[...] + g0_ref[...]) * d + b_ref[:, :128]
    t1 = (a1_ref[...] + g1_ref[...]) * d + b_ref[:, 128:]
    h = jnp.maximum(jnp.concatenate([t0, t1], axis=1), 0.0)
    ids = bat_ref[0]
    onehot = (lax.broadcasted_iota(_i32, (GG, 128), 0) == ids).astype(_f32)

    @pl.when(i == 0)
    def _():
        sums[...] = jnp.zeros((GG, HH), _f32)
        cnts[...] = jnp.zeros((GG, 128), _f32)

    sums[...] += jnp.dot(onehot, h, preferred_element_type=_f32)
    cnts[...] += jnp.dot(onehot, jnp.ones((128, 128), _f32),
                         preferred_element_type=_f32)

    @pl.when(i == 78)
    def _():
        c = jnp.maximum(cnts[...], 1.0)
        pooled = jnp.concatenate([sums[:, :128] / c, sums[:, 128:] / c],
                                 axis=1)
        out_ref[...] = (jnp.dot(pooled, wfc_ref[...],
                                preferred_element_type=_f32)
                        + bfc_ref[...])


def _pool_call(a0, a1, g0, g1, drep, b2d, bat3, wfc, bfc2d):
    return pl.pallas_call(
        _pool_body,
        grid=(79,),
        in_specs=[
            pl.BlockSpec((128, 128), lambda i: (i, 0)),
            pl.BlockSpec((128, 128), lambda i: (i, 0)),
            pl.BlockSpec((128, 128), lambda i: (i, 0)),
            pl.BlockSpec((128, 128), lambda i: (i, 0)),
            pl.BlockSpec((128, 128), lambda i: (i, 0)),
            pl.BlockSpec((1, HH), lambda i: (0, 0)),
            pl.BlockSpec((1, 1, 128), lambda i: (i, 0, 0)),
            pl.BlockSpec((HH, 128), lambda i: (0, 0)),
            pl.BlockSpec((1, 128), lambda i: (0, 0)),
        ],
        out_specs=pl.BlockSpec((GG, 128), lambda i: (0, 0)),
        out_shape=jax.ShapeDtypeStruct((GG, 128), _f32),
        scratch_shapes=[
            pltpu.VMEM((GG, HH), _f32),
            pltpu.VMEM((GG, 128), _f32),
        ],
    )(a0, a1, g0, g1, drep, b2d, bat3, wfc, bfc2d)


def kernel(x, edge_index, batch, W1, b1, W2, b2, W3, b3, Wfc, bfc):
    src = edge_index[0]
    dst = edge_index[1]
    padv = jnp.full((EPAD - EE,), NN, _i32)
    src_t = jnp.concatenate([src, padv]).reshape(16, NB, 128)
    dst_t = jnp.concatenate([dst, padv]).reshape(16, NB, 128)
    src_c = src_t.reshape(16 * NCH, CH, 128)
    dst_c = dst_t.reshape(16 * NCH, CH, 128)

    xp = jnp.pad(x, ((0, NPAD - NN), (0, 0)))
    bat3 = jnp.pad(batch, (0, NPAD - NN), constant_values=GG).reshape(
        79, 1, 128)
    wfc_p = jnp.pad(Wfc, ((0, 0), (0, 128 - CC)))
    bfc_p = jnp.pad(bfc, (0, 128 - CC)).reshape(1, 128)

    cnt0, cnt1 = _deg_kernel(dst_c)
    drep = _dinv_call(cnt0, cnt1)

    g0, g1 = _lin1_call(xp, W1, drep)
    a0, a1 = _agg_kernel(g0, g1, src_c, dst_c)
    g0, g1 = _mid_call(a0, a1, g0, g1, drep, W2, b1.reshape(1, HH))
    a0, a1 = _agg_kernel(g0, g1, src_c, dst_c)
    g0, g1 = _mid_call(a0, a1, g0, g1, drep, W3, b2.reshape(1, HH))
    a0, a1 = _agg_kernel(g0, g1, src_c, dst_c)
    out = _pool_call(a0, a1, g0, g1, drep, b3.reshape(1, HH), bat3,
                     wfc_p, bfc_p)
    return out[:, :CC]


# speedup vs baseline: 7.9949x; 1.2028x over previous
import functools

import jax
import jax.numpy as jnp
from jax import lax
from jax.experimental import pallas as pl
from jax.experimental.pallas import tpu as pltpu
from jax.experimental.pallas import tpu_sc as plsc

NN = 10000
EE = 320000
FIN = 128
HH = 256
CC = 10
GG = 64

NPAD = 79 * 128
NB = 160
CH = 32
NCH = NB // CH
EPT = NB * 128
EPAD = 16 * EPT
RPT = NPAD // 16

_mesh = plsc.VectorSubcoreMesh(core_axis_name="c", subcore_axis_name="s")

_f32 = jnp.float32
_i32 = jnp.int32


@functools.partial(
    pl.kernel,
    mesh=_mesh,
    out_type=[
        jax.ShapeDtypeStruct((NPAD, 128), _f32),
        jax.ShapeDtypeStruct((NPAD, 128), _f32),
    ],
    scratch_types=[
        pltpu.VMEM((CH, 128), _i32),
        pltpu.VMEM((128,), _i32),
        pltpu.VMEM((128, 128), _f32),
        pltpu.VMEM((128, 128), _f32),
        pltpu.VMEM_SHARED((NPAD, 128), _f32),
        pltpu.SemaphoreType.DMA,
    ],
)
def _deg_kernel(dst_hbm, cnt0, cnt1, dstbuf, didx, onesbuf, zbuf, shared,
                sem):
    cid = lax.axis_index("c")
    sid = lax.axis_index("s")

    def orow(k, carry):
        def ocol(l, c2):
            onesbuf[k, pl.ds(l * 16, 16)] = jnp.ones((16,), _f32)
            zbuf[k, pl.ds(l * 16, 16)] = jnp.zeros((16,), _f32)
            return c2

        lax.fori_loop(0, 8, ocol, 0)
        return carry

    lax.fori_loop(0, 128, orow, 0)
    base = sid * RPT
    for k in range(4):
        pltpu.sync_copy(zbuf, shared.at[pl.ds(base + k * 128, 128)])
    pltpu.sync_copy(zbuf.at[pl.ds(0, RPT - 512)],
                    shared.at[pl.ds(base + 512, RPT - 512)])
    plsc.subcore_barrier()

    def chunk(c, carry):
        row = sid * NCH + c
        pltpu.sync_copy(dst_hbm.at[row], dstbuf)

        def body(j, carry2):
            def stage(l, c2):
                didx[pl.ds(l * 16, 16)] = dstbuf[j, pl.ds(l * 16, 16)]
                return c2

            lax.fori_loop(0, 8, stage, 0)
            pltpu.sync_copy(onesbuf, shared.at[didx], add=True)
            return carry2

        lax.fori_loop(cid * (CH // 2), cid * (CH // 2) + CH // 2, body, 0)
        return carry

    lax.fori_loop(0, NCH, chunk, 0)
    plsc.subcore_barrier()

    def wout(dsthbm):
        for k in range(5):
            rows = 128 if k < 4 else RPT - 512
            off = base + k * 128
            pltpu.sync_copy(shared.at[pl.ds(off, rows)],
                            zbuf.at[pl.ds(0, rows)])
            pltpu.sync_copy(zbuf.at[pl.ds(0, rows)],
                            dsthbm.at[pl.ds(off, rows)])

    @pl.when(cid == 0)
    def _():
        wout(cnt0)

    @pl.when(cid == 1)
    def _():
        wout(cnt1)


@functools.partial(
    pl.kernel,
    mesh=_mesh,
    out_type=[
        jax.ShapeDtypeStruct((NPAD, 128), _f32),
        jax.ShapeDtypeStruct((NPAD, 128), _f32),
    ],
    scratch_types=[
        pltpu.VMEM((CH, 128), _i32),
        pltpu.VMEM((CH, 128), _i32),
        pltpu.VMEM((128,), _i32),
        pltpu.VMEM((128,), _i32),
        pltpu.VMEM((128,), _i32),
        pltpu.VMEM((128,), _i32),
        pltpu.VMEM((128, 128), _f32),
        pltpu.VMEM((128, 128), _f32),
        pltpu.VMEM_SHARED((NPAD, 128), _f32),
        pltpu.SemaphoreType.DMA,
        pltpu.SemaphoreType.DMA,
    ],
)
def _agg_kernel(g0, g1, src_hbm, dst_hbm, acc0, acc1,
                srcbuf, dstbuf, sidxA, didxA, sidxB, didxB, gbuf, gbuf2,
                shared, semA, semB):
    cid = lax.axis_index("c")
    sid = lax.axis_index("s")

    def zrow(k, carry):
        gbuf[k >> 3, pl.ds((k & 7) * 16, 16)] = jnp.zeros((16,), _f32)
        return carry

    lax.fori_loop(0, 128 * 8, zrow, 0)
    base = sid * RPT
    for k in range(4):
        pltpu.sync_copy(gbuf, shared.at[pl.ds(base + k * 128, 128)])
    pltpu.sync_copy(gbuf.at[pl.ds(0, RPT - 512)],
                    shared.at[pl.ds(base + 512, RPT - 512)])
    plsc.subcore_barrier()

    def run(gtab):
        def stage(j, sx, dx):
            def st(l, c2):
                sx[pl.ds(l * 16, 16)] = srcbuf[j, pl.ds(l * 16, 16)]
                dx[pl.ds(l * 16, 16)] = dstbuf[j, pl.ds(l * 16, 16)]
                return c2

            lax.fori_loop(0, 8, st, 0)

        def chunk(c, carry):
            row = sid * NCH + c
            pltpu.sync_copy(src_hbm.at[row], srcbuf)
            pltpu.sync_copy(dst_hbm.at[row], dstbuf)
            stage(0, sidxA, didxA)
            pltpu.async_copy(gtab.at[sidxA], gbuf, semA)

            def pair(t, carry2):
                j1 = 2 * t + 1
                stage(j1, sidxB, didxB)
                pltpu.async_copy(gtab.at[sidxB], gbuf2, semB)
                pltpu.make_async_copy(gtab.at[sidxA], gbuf, semA).wait()
                pltpu.sync_copy(gbuf, shared.at[didxA], add=True)

                @pl.when(t < CH // 2 - 1)
                def _():
                    stage(j1 + 1, sidxA, didxA)
                    pltpu.async_copy(gtab.at[sidxA], gbuf, semA)

                pltpu.make_async_copy(gtab.at[sidxB], gbuf2, semB).wait()
                pltpu.sync_copy(gbuf2, shared.at[didxB], add=True)
                return carry2

            lax.fori_loop(0, CH // 2, pair, 0)
            return carry

        lax.fori_loop(0, NCH, chunk, 0)

    @pl.when(cid == 0)
    def _():
        run(g0)

    @pl.when(cid == 1)
    def _():
        run(g1)

    plsc.subcore_barrier()

    def wout(dsthbm):
        for k in range(5):
            rows = 128 if k < 4 else RPT - 512
            off = base + k * 128
            pltpu.sync_copy(shared.at[pl.ds(off, rows)],
                            gbuf.at[pl.ds(0, rows)])
            pltpu.sync_copy(gbuf.at[pl.ds(0, rows)],
                            dsthbm.at[pl.ds(off, rows)])

    @pl.when(cid == 0)
    def _():
        wout(acc0)

    @pl.when(cid == 1)
    def _():
        wout(acc1)


def _dinv_body(c0_ref, c1_ref, out_ref):
    i = pl.program_id(0)
    cnt = c0_ref[:, :1] + c1_ref[:, :1]
    node = i * 128 + lax.broadcasted_iota(_i32, (128, 1), 0)
    dinv = jnp.where(node < NN, lax.rsqrt(cnt + 1.0), 0.0)
    out_ref[...] = jnp.broadcast_to(dinv, (128, 128))


def _dinv_call(cnt0, cnt1):
    return pl.pallas_call(
        _dinv_body,
        grid=(79,),
        in_specs=[
            pl.BlockSpec((128, 128), lambda i: (i, 0)),
            pl.BlockSpec((128, 128), lambda i: (i, 0)),
        ],
        out_specs=pl.BlockSpec((128, 128), lambda i: (i, 0)),
        out_shape=jax.ShapeDtypeStruct((NPAD, 128), _f32),
    )(cnt0, cnt1)


def _lin1_body(x_ref, w_ref, drep_ref, g0_ref, g1_ref):
    hw = jnp.dot(x_ref[...], w_ref[...], preferred_element_type=_f32)
    d = drep_ref[...]
    g0_ref[...] = hw[:, :128] * d
    g1_ref[...] = hw[:, 128:] * d


def _lin1_call(xp, w, drep):
    return pl.pallas_call(
        _lin1_body,
        grid=(79,),
        in_specs=[
            pl.BlockSpec((128, FIN), lambda i: (i, 0)),
            pl.BlockSpec((FIN, HH), lambda i: (0, 0)),
            pl.BlockSpec((128, 128), lambda i: (i, 0)),
        ],
        out_specs=[
            pl.BlockSpec((128, 128), lambda i: (i, 0)),
            pl.BlockSpec((128, 128), lambda i: (i, 0)),
        ],
        out_shape=[
            jax.ShapeDtypeStruct((NPAD, 128), _f32),
            jax.ShapeDtypeStruct((NPAD, 128), _f32),
        ],
    )(xp, w, drep)


def _mid_body(a0_ref, a1_ref, g0_ref, g1_ref, drep_ref, w_ref, b_ref,
              o0_ref, o1_ref):
    d = drep_ref[...]
    t0 = (a0_ref[...] + g0_ref[...]) * d + b_ref[:, :128]
    t1 = (a1_ref[...] + g1_ref[...]) * d + b_ref[:, 128:]
    h = jnp.maximum(jnp.concatenate([t0, t1], axis=1), 0.0)
    hw = jnp.dot(h, w_ref[...], preferred_element_type=_f32)
    o0_ref[...] = hw[:, :128] * d
    o1_ref[...] = hw[:, 128:] * d


def _mid_call(a0, a1, g0, g1, drep, w, b2d):
    return pl.pallas_call(
        _mid_body,
        grid=(79,),
        in_specs=[
            pl.BlockSpec((128, 128), lambda i: (i, 0)),
            pl.BlockSpec((128, 128), lambda i: (i, 0)),
            pl.BlockSpec((128, 128), lambda i: (i, 0)),
            pl.BlockSpec((128, 128), lambda i: (i, 0)),
            pl.BlockSpec((128, 128), lambda i: (i, 0)),
            pl.BlockSpec((HH, HH), lambda i: (0, 0)),
            pl.BlockSpec((1, HH), lambda i: (0, 0)),
        ],
        out_specs=[
            pl.BlockSpec((128, 128), lambda i: (i, 0)),
            pl.BlockSpec((128, 128), lambda i: (i, 0)),
        ],
        out_shape=[
            jax.ShapeDtypeStruct((NPAD, 128), _f32),
            jax.ShapeDtypeStruct((NPAD, 128), _f32),
        ],
    )(a0, a1, g0, g1, drep, w, b2d)


def _pool_body(a0_ref, a1_ref, g0_ref, g1_ref, drep_ref, b_ref, bat_ref,
               wfc_ref, bfc_ref, out_ref, sums, cnts):
    i = pl.program_id(0)
    d = drep_ref[...]
    t0 = (a0_ref[...] + g0_ref[...]) * d + b_ref[:, :128]
    t1 = (a1_ref[...] + g1_ref[...]) * d + b_ref[:, 128:]
    h = jnp.maximum(jnp.concatenate([t0, t1], axis=1), 0.0)
    ids = bat_ref[0]
    onehot = (lax.broadcasted_iota(_i32, (GG, 128), 0) == ids).astype(_f32)

    @pl.when(i == 0)
    def _():
        sums[...] = jnp.zeros((GG, HH), _f32)
        cnts[...] = jnp.zeros((GG, 128), _f32)

    sums[...] += jnp.dot(onehot, h, preferred_element_type=_f32)
    cnts[...] += jnp.dot(onehot, jnp.ones((128, 128), _f32),
                         preferred_element_type=_f32)

    @pl.when(i == 78)
    def _():
        c = jnp.maximum(cnts[...], 1.0)
        pooled = jnp.concatenate([sums[:, :128] / c, sums[:, 128:] / c],
                                 axis=1)
        out_ref[...] = (jnp.dot(pooled, wfc_ref[...],
                                preferred_element_type=_f32)
                        + bfc_ref[...])


def _pool_call(a0, a1, g0, g1, drep, b2d, bat3, wfc, bfc2d):
    return pl.pallas_call(
        _pool_body,
        grid=(79,),
        in_specs=[
            pl.BlockSpec((128, 128), lambda i: (i, 0)),
            pl.BlockSpec((128, 128), lambda i: (i, 0)),
            pl.BlockSpec((128, 128), lambda i: (i, 0)),
            pl.BlockSpec((128, 128), lambda i: (i, 0)),
            pl.BlockSpec((128, 128), lambda i: (i, 0)),
            pl.BlockSpec((1, HH), lambda i: (0, 0)),
            pl.BlockSpec((1, 1, 128), lambda i: (i, 0, 0)),
            pl.BlockSpec((HH, 128), lambda i: (0, 0)),
            pl.BlockSpec((1, 128), lambda i: (0, 0)),
        ],
        out_specs=pl.BlockSpec((GG, 128), lambda i: (0, 0)),
        out_shape=jax.ShapeDtypeStruct((GG, 128), _f32),
        scratch_shapes=[
            pltpu.VMEM((GG, HH), _f32),
            pltpu.VMEM((GG, 128), _f32),
        ],
    )(a0, a1, g0, g1, drep, b2d, bat3, wfc, bfc2d)


def kernel(x, edge_index, batch, W1, b1, W2, b2, W3, b3, Wfc, bfc):
    src = edge_index[0]
    dst = edge_index[1]
    padv = jnp.full((EPAD - EE,), NN, _i32)
    src_t = jnp.concatenate([src, padv]).reshape(16, NB, 128)
    dst_t = jnp.concatenate([dst, padv]).reshape(16, NB, 128)
    src_c = src_t.reshape(16 * NCH, CH, 128)
    dst_c = dst_t.reshape(16 * NCH, CH, 128)

    xp = jnp.pad(x, ((0, NPAD - NN), (0, 0)))
    bat3 = jnp.pad(batch, (0, NPAD - NN), constant_values=GG).reshape(
        79, 1, 128)
    wfc_p = jnp.pad(Wfc, ((0, 0), (0, 128 - CC)))
    bfc_p = jnp.pad(bfc, (0, 128 - CC)).reshape(1, 128)

    cnt0, cnt1 = _deg_kernel(dst_c)
    drep = _dinv_call(cnt0, cnt1)

    g0, g1 = _lin1_call(xp, W1, drep)
    a0, a1 = _agg_kernel(g0, g1, src_c, dst_c)
    g0, g1 = _mid_call(a0, a1, g0, g1, drep, W2, b1.reshape(1, HH))
    a0, a1 = _agg_kernel(g0, g1, src_c, dst_c)
    g0, g1 = _mid_call(a0, a1, g0, g1, drep, W3, b2.reshape(1, HH))
    a0, a1 = _agg_kernel(g0, g1, src_c, dst_c)
    out = _pool_call(a0, a1, g0, g1, drep, b3.reshape(1, HH), bat3,
                     wfc_p, bfc_p)
    return out[:, :CC]
